# baseline (speedup 1.0000x reference)
import functools

import jax
import jax.numpy as jnp
from jax import lax
from jax.experimental import pallas as pl
from jax.experimental.pallas import tpu as pltpu
from jax.experimental.pallas import tpu_sc as plsc

ENTITY_NUM = 1000000
DIM = 64
BATCH = 16384

NC = 2
NS = 16
NW = NC * NS
BPW = BATCH // NW
CHUNK = 128
NCHUNK = BPW // CHUNK


def _body(ent_hbm, rel_hbm, ph_hbm, pr_hbm, pt_hbm, nh_hbm, nr_hbm, nt_hbm,
          out_phr, out_pt, out_nhr, out_nt,
          idx_a, idx_b, rows_a, rows_b, sem_a, sem_b):
    wid = lax.axis_index("s") * NC + lax.axis_index("c")
    base = wid * BPW

    def add_rows(_i, _):
        for j in range(DIM // 16):
            sl = pl.ds(j * 16, 16)
            rows_a[_i, sl] = rows_a[_i, sl] + rows_b[_i, sl]
        return ()

    def do_pair(idx1_hbm, idx2_hbm, out_hbm):
        for c in range(NCHUNK):
            off = base + c * CHUNK
            pltpu.sync_copy(idx1_hbm.at[pl.ds(off, CHUNK)], idx_a)
            pltpu.sync_copy(idx2_hbm.at[pl.ds(off, CHUNK)], idx_b)
            cp_a = pltpu.async_copy(ent_hbm.at[idx_a], rows_a, sem_a)
            cp_b = pltpu.async_copy(rel_hbm.at[idx_b], rows_b, sem_b)
            cp_a.wait()
            cp_b.wait()
            lax.fori_loop(0, CHUNK, add_rows, (), unroll=4)
            pltpu.sync_copy(rows_a, out_hbm.at[pl.ds(off, CHUNK)])

    def do_single(idx_hbm, out_hbm):
        for c in range(NCHUNK):
            off = base + c * CHUNK
            pltpu.sync_copy(idx_hbm.at[pl.ds(off, CHUNK)], idx_a)
            pltpu.async_copy(ent_hbm.at[idx_a], rows_a, sem_a).wait()
            pltpu.sync_copy(rows_a, out_hbm.at[pl.ds(off, CHUNK)])

    do_pair(ph_hbm, pr_hbm, out_phr)
    do_single(pt_hbm, out_pt)
    do_pair(nh_hbm, nr_hbm, out_nhr)
    do_single(nt_hbm, out_nt)


@jax.jit
def kernel(entity_emb, relation_emb, pos_head, pos_relation, pos_tail,
           neg_head, neg_relation, neg_tail):
    out_sds = jax.ShapeDtypeStruct((BATCH, DIM), jnp.float32)
    mesh = plsc.VectorSubcoreMesh(
        core_axis_name="c", subcore_axis_name="s", num_cores=NC, num_subcores=NS)
    f = pl.kernel(
        _body,
        out_type=(out_sds, out_sds, out_sds, out_sds),
        mesh=mesh,
        compiler_params=pltpu.CompilerParams(use_tc_tiling_on_sc=False),
        scratch_types=[
            pltpu.VMEM((CHUNK,), jnp.int32),
            pltpu.VMEM((CHUNK,), jnp.int32),
            pltpu.VMEM((CHUNK, DIM), jnp.float32),
            pltpu.VMEM((CHUNK, DIM), jnp.float32),
            pltpu.SemaphoreType.DMA,
            pltpu.SemaphoreType.DMA,
        ],
    )
    return f(entity_emb, relation_emb,
             pos_head.astype(jnp.int32), pos_relation.astype(jnp.int32),
             pos_tail.astype(jnp.int32), neg_head.astype(jnp.int32),
             neg_relation.astype(jnp.int32), neg_tail.astype(jnp.int32))


# speedup vs baseline: 1.5175x; 1.5175x over previous
import functools

import jax
import jax.numpy as jnp
from jax import lax
from jax.experimental import pallas as pl
from jax.experimental.pallas import tpu as pltpu
from jax.experimental.pallas import tpu_sc as plsc

ENTITY_NUM = 1000000
DIM = 64
BATCH = 16384

NC = 2
NS = 16
NW = NC * NS
BPW = BATCH // NW
CHUNK = 128
NCHUNK = BPW // CHUNK


def _body(ent_hbm, rel_hbm, ph_hbm, pr_hbm, pt_hbm, nh_hbm, nr_hbm, nt_hbm,
          out_phr, out_pt, out_nhr, out_nt,
          idx_a, idx_b, rows_a, rows_b, sem_a, sem_b):
    wid = lax.axis_index("s") * NC + lax.axis_index("c")
    base = wid * BPW

    def fire_rows(table_hbm, idx_ref, rows_ref, sem):
        def one(g, _):
            v = idx_ref[pl.ds(g * 16, 16)]
            for j in range(16):
                s = v[j]
                pltpu.async_copy(table_hbm.at[pl.ds(s, 1)],
                                 rows_ref.at[pl.ds(g * 16 + j, 1)], sem)
            return ()
        lax.fori_loop(0, CHUNK // 16, one, ())

    def drain(rows_ref, sem):
        pltpu.make_async_copy(ent_hbm.at[pl.ds(0, CHUNK)], rows_ref, sem).wait()

    def add_rows(i, _):
        for j in range(DIM // 16):
            sl = pl.ds(j * 16, 16)
            rows_a[i, sl] = rows_a[i, sl] + rows_b[i, sl]
        return ()

    def do_pair(idx1_hbm, idx2_hbm, out_hbm):
        for c in range(NCHUNK):
            off = base + c * CHUNK
            pltpu.sync_copy(idx1_hbm.at[pl.ds(off, CHUNK)], idx_a)
            pltpu.sync_copy(idx2_hbm.at[pl.ds(off, CHUNK)], idx_b)
            fire_rows(ent_hbm, idx_a, rows_a, sem_a)
            fire_rows(rel_hbm, idx_b, rows_b, sem_b)
            drain(rows_a, sem_a)
            drain(rows_b, sem_b)
            lax.fori_loop(0, CHUNK, add_rows, (), unroll=4)
            pltpu.sync_copy(rows_a, out_hbm.at[pl.ds(off, CHUNK)])

    def do_single(idx_hbm, out_hbm):
        for c in range(NCHUNK):
            off = base + c * CHUNK
            pltpu.sync_copy(idx_hbm.at[pl.ds(off, CHUNK)], idx_a)
            fire_rows(ent_hbm, idx_a, rows_a, sem_a)
            drain(rows_a, sem_a)
            pltpu.sync_copy(rows_a, out_hbm.at[pl.ds(off, CHUNK)])

    do_pair(ph_hbm, pr_hbm, out_phr)
    do_single(pt_hbm, out_pt)
    do_pair(nh_hbm, nr_hbm, out_nhr)
    do_single(nt_hbm, out_nt)


@jax.jit
def kernel(entity_emb, relation_emb, pos_head, pos_relation, pos_tail,
           neg_head, neg_relation, neg_tail):
    out_sds = jax.ShapeDtypeStruct((BATCH, DIM), jnp.float32)
    mesh = plsc.VectorSubcoreMesh(
        core_axis_name="c", subcore_axis_name="s", num_cores=NC, num_subcores=NS)
    f = pl.kernel(
        _body,
        out_type=(out_sds, out_sds, out_sds, out_sds),
        mesh=mesh,
        scratch_types=[
            pltpu.VMEM((CHUNK,), jnp.int32),
            pltpu.VMEM((CHUNK,), jnp.int32),
            pltpu.VMEM((CHUNK, DIM), jnp.float32),
            pltpu.VMEM((CHUNK, DIM), jnp.float32),
            pltpu.SemaphoreType.DMA,
            pltpu.SemaphoreType.DMA,
        ],
    )
    return f(entity_emb, relation_emb,
             pos_head.astype(jnp.int32), pos_relation.astype(jnp.int32),
             pos_tail.astype(jnp.int32), neg_head.astype(jnp.int32),
             neg_relation.astype(jnp.int32), neg_tail.astype(jnp.int32))
